# Initial kernel scaffold; baseline (speedup 1.0000x reference)
import jax
import jax.numpy as jnp
from jax.experimental import pallas as pl


def kernel(x, edge_index, edge_type, W1, b1, W2, b2):
    raise NotImplementedError("write your pallas kernel here")


# speedup vs baseline: 7.1063x; 7.1063x over previous
import functools

import jax
import jax.numpy as jnp
from jax import lax
from jax.experimental import pallas as pl
from jax.experimental.pallas import tpu as pltpu
from jax.experimental.pallas import tpu_sc as plsc

D = 256
R = 4
HALF = 128
NC = 2
NS = 16
L = 16
K = 128
NP = 10112
RT = NP // NS


def _mm1_body(x_ref, w_ref, o_ref):
    o_ref[...] = jnp.dot(x_ref[...], w_ref[...],
                         preferred_element_type=jnp.float32)


def _mm2_body(a_ref, w_ref, o_ref):
    h0 = jnp.maximum(a_ref[0], 0.0)
    h1 = jnp.maximum(a_ref[1], 0.0)
    o_ref[...] = (
        jnp.dot(h0, w_ref[pl.ds(0, HALF), :], preferred_element_type=jnp.float32)
        + jnp.dot(h1, w_ref[pl.ds(HALF, HALF), :], preferred_element_type=jnp.float32))


def _mm1(x, wt, bn=1000):
    n = x.shape[0]
    return pl.pallas_call(
        _mm1_body,
        grid=(n // bn,),
        in_specs=[pl.BlockSpec((bn, D), lambda i: (i, 0)),
                  pl.BlockSpec((D, NC * R * HALF), lambda i: (0, 0))],
        out_specs=pl.BlockSpec((bn, NC * R * HALF), lambda i: (i, 0)),
        out_shape=jax.ShapeDtypeStruct((n, NC * R * HALF), jnp.float32),
    )(x, wt)


def _mm2(agg, wt, n, bn=1000):
    return pl.pallas_call(
        _mm2_body,
        grid=(n // bn,),
        in_specs=[pl.BlockSpec((NC, bn, HALF), lambda i: (0, i, 0)),
                  pl.BlockSpec((D, NC * R * HALF), lambda i: (0, 0))],
        out_specs=pl.BlockSpec((bn, NC * R * HALF), lambda i: (i, 0)),
        out_shape=jax.ShapeDtypeStruct((n, NC * R * HALF), jnp.float32),
    )(agg, wt)


def _sc_aggregate(table, srcp, dstp, relp, binit, tp, ch):
    mesh = plsc.VectorSubcoreMesh(core_axis_name="c", subcore_axis_name="s")

    @functools.partial(
        pl.kernel, mesh=mesh,
        out_type=jax.ShapeDtypeStruct((NC * NP, HALF), jnp.float32),
        scratch_types=[
            pltpu.VMEM_SHARED((NP, HALF), jnp.float32),
            pltpu.VMEM((tp,), jnp.int32),
            pltpu.VMEM((tp,), jnp.int32),
            pltpu.VMEM((tp,), jnp.int32),
            pltpu.VMEM((K,), jnp.int32),
            pltpu.VMEM((K,), jnp.int32),
            pltpu.VMEM((K, HALF), jnp.float32),
            pltpu.SemaphoreType.DMA,
        ],
    )
    def k(table_h, src_h, dst_h, rel_h, binit_h, out_h,
          acc, src_v, dst_v, rel_v, cidx_v, didx_v, rows_v, sem):
        c = lax.axis_index("c")
        s = lax.axis_index("s")
        r0 = s * RT
        pltpu.sync_copy(binit_h.at[pl.ds(c * NP + r0, RT)], acc.at[pl.ds(r0, RT)])
        e0 = s * tp
        pltpu.sync_copy(src_h.at[pl.ds(e0, tp)], src_v)
        pltpu.sync_copy(dst_h.at[pl.ds(e0, tp)], dst_v)
        pltpu.sync_copy(rel_h.at[pl.ds(e0, tp)], rel_v)
        plsc.subcore_barrier()

        def chunk(j, carry):
            base = j * K
            for i in range(K // L):
                sl = pl.ds(base + i * L, L)
                cidx_v[pl.ds(i * L, L)] = (src_v[sl] * (R * NC)
                                           + rel_v[sl] * NC + c)
                didx_v[pl.ds(i * L, L)] = dst_v[sl]
            pltpu.async_copy(table_h.at[cidx_v], rows_v, sem).wait()
            pltpu.sync_copy(rows_v, acc.at[didx_v], add=True)
            return carry

        lax.fori_loop(0, ch, chunk, 0)
        plsc.subcore_barrier()
        pltpu.sync_copy(acc.at[pl.ds(r0, RT)], out_h.at[pl.ds(c * NP + r0, RT)])

    return k(table, srcp, dstp, relp, binit)


def kernel(x, edge_index, edge_type, W1, b1, W2, b2):
    n = x.shape[0]
    e = edge_index.shape[1]
    src = edge_index[0]
    dst = edge_index[1]

    ch = -(-e // (NS * K))
    tp = ch * K
    padn = NS * tp - e
    srcp = jnp.concatenate([src, jnp.zeros((padn,), jnp.int32)])
    dstp = jnp.concatenate([dst, jnp.full((padn,), n, jnp.int32)])
    relp = jnp.concatenate([edge_type, jnp.zeros((padn,), jnp.int32)])

    w1t = jnp.transpose(W1, (2, 0, 1)).reshape(D, R * D)
    w2t = jnp.transpose(W2, (2, 0, 1)).reshape(D, R * D)
    binit1 = jnp.broadcast_to(b1.reshape(NC, 1, HALF), (NC, NP, HALF)).reshape(NC * NP, HALF)
    binit2 = jnp.broadcast_to(b2.reshape(NC, 1, HALF), (NC, NP, HALF)).reshape(NC * NP, HALF)

    t1 = _mm1(x, w1t).reshape(n * R * NC, HALF)
    agg1 = _sc_aggregate(t1, srcp, dstp, relp, binit1, tp, ch)
    t2 = _mm2(agg1.reshape(NC, NP, HALF), w2t, n).reshape(n * R * NC, HALF)
    agg2 = _sc_aggregate(t2, srcp, dstp, relp, binit2, tp, ch)
    a2 = agg2.reshape(NC, NP, HALF)
    return jnp.concatenate([a2[0, :n], a2[1, :n]], axis=1)
